# baseline (speedup 1.0000x reference)
import functools

import jax
import jax.numpy as jnp
from jax import lax
from jax.experimental import pallas as pl
from jax.experimental.pallas import tpu as pltpu
from jax.experimental.pallas import tpu_sc as plsc

D = 128
B = 16384
NC = 2
NS = 16
NW = NC * NS
BPW = B // NW
CH = 32
NCHUNK = BPW // CH
NSL = D // 16
NR = 1000

_GATHER_DNUMS = lax.GatherDimensionNumbers(
    offset_dims=(), collapsed_slice_dims=(0,), start_index_map=(0,))


def _lane_permute(v, idx):
    return lax.gather(v, idx[:, None], _GATHER_DNUMS, slice_sizes=(1,),
                      mode=lax.GatherScatterMode.PROMISE_IN_BOUNDS)


@functools.partial(
    pl.kernel,
    out_type=jax.ShapeDtypeStruct((B,), jnp.float32),
    mesh=plsc.VectorSubcoreMesh(core_axis_name="c", subcore_axis_name="s"),
    scratch_types=[
        pltpu.VMEM((BPW,), jnp.int32),
        pltpu.VMEM((NCHUNK, CH), jnp.int32),
        pltpu.VMEM((BPW,), jnp.int32),
        pltpu.VMEM((2, CH, D), jnp.float32),
        pltpu.VMEM((2, CH, D), jnp.float32),
        pltpu.VMEM((2, CH, D), jnp.float32),
        pltpu.VMEM((2, CH, D), jnp.float32),
        pltpu.VMEM((2, CH, D), jnp.float32),
        pltpu.VMEM((2, CH, D), jnp.float32),
        pltpu.VMEM((BPW,), jnp.float32),
        pltpu.VMEM_SHARED((NR, D), jnp.float32),
        pltpu.VMEM_SHARED((NR, D), jnp.float32),
        pltpu.SemaphoreType.DMA,
        pltpu.SemaphoreType.DMA,
        pltpu.SemaphoreType.DMA,
        pltpu.SemaphoreType.DMA,
        pltpu.SemaphoreType.DMA,
        pltpu.SemaphoreType.DMA,
    ],
)
def _sc_complex_score(e1_hbm, e2_hbm, r1_hbm, r2_hbm,
                      heads_hbm, rels_hbm, tails_hbm, out_hbm,
                      h_v, r_v, t_v, e11, e12, e21, e22, rr1, rr2,
                      sc_v, r1_sh, r2_sh, sem0, sem1, sem2, sem3,
                      isem, ssem):
    wid = lax.axis_index("s") * NC + lax.axis_index("c")
    sid = lax.axis_index("s")
    base = wid * BPW

    @pl.when(sid == 0)
    def _():
        pltpu.async_copy(r1_hbm, r1_sh, ssem)

    @pl.when(sid == 1)
    def _():
        pltpu.async_copy(r2_hbm, r2_sh, ssem)

    icps = [
        pltpu.async_copy(heads_hbm.at[pl.ds(base, BPW)], h_v, isem),
        pltpu.async_copy(tails_hbm.at[pl.ds(base, BPW)], t_v, isem),
    ]
    icps += [
        pltpu.async_copy(rels_hbm.at[pl.ds(base + cc * CH, CH)], r_v.at[cc],
                         isem)
        for cc in range(NCHUNK)
    ]
    for cp in icps:
        cp.wait()

    sems = (sem0, sem1)
    rsems = (sem2, sem3)

    def fire_ent(c, slot):
        off = c * CH
        hs = h_v.at[pl.ds(off, CH)]
        ts = t_v.at[pl.ds(off, CH)]
        sem = sems[slot]
        pltpu.async_copy(e1_hbm.at[hs], e11.at[slot], sem)
        pltpu.async_copy(e2_hbm.at[hs], e12.at[slot], sem)
        pltpu.async_copy(e1_hbm.at[ts], e21.at[slot], sem)
        pltpu.async_copy(e2_hbm.at[ts], e22.at[slot], sem)

    def fire_rel(c, slot):
        rs = r_v.at[c]
        sem = rsems[slot]
        pltpu.async_copy(r1_sh.at[rs], rr1.at[slot], sem)
        pltpu.async_copy(r2_sh.at[rs], rr2.at[slot], sem)

    def fire(c, slot):
        fire_ent(c, slot)
        fire_rel(c, slot)

    def drain(slot, c):
        off = c * CH
        sem = sems[slot]
        for dst in (e11, e12, e21, e22):
            pltpu.make_async_copy(
                e1_hbm.at[pl.ds(0, CH)], dst.at[slot], sem).wait()
        rs = r_v.at[c]
        rsem = rsems[slot]
        pltpu.make_async_copy(r1_sh.at[rs], rr1.at[slot], rsem).wait()
        pltpu.make_async_copy(r2_sh.at[rs], rr2.at[slot], rsem).wait()

    def compute(c, slot):
        off = c * CH
        b11, b12 = e11.at[slot], e12.at[slot]
        b21, b22 = e21.at[slot], e22.at[slot]
        c1, c2 = rr1.at[slot], rr2.at[slot]

        def group_body(g, rc):
            lane = lax.iota(jnp.int32, 16)
            perms = [lane ^ k for k in (1, 2, 4, 8)]

            @plsc.parallel_loop(0, 16, 1, unroll=8,
                                carry=jnp.zeros((16,), jnp.float32))
            def w_loop(j, w):
                i = g * 16 + j
                accp = jnp.zeros((16,), jnp.float32)
                accq = jnp.zeros((16,), jnp.float32)
                for s in range(NSL):
                    sl = pl.ds(s * 16, 16)
                    a = b11[i, sl]
                    b = b12[i, sl]
                    u = b21[i, sl]
                    v = b22[i, sl]
                    p1 = c1[i, sl]
                    p2 = c2[i, sl]
                    accp = accp + p1 * (a * u + b * v)
                    accq = accq + p2 * (a * v - b * u)
                t = accp + accq
                for p in perms:
                    t = t + _lane_permute(t, p)
                return jnp.where(lane == j, t, w)

            sc_v[pl.ds(off + g * 16, 16)] = 1.0 / (1.0 + jnp.exp(-w_loop))
            return rc

        lax.fori_loop(0, CH // 16, group_body, 0)

    fire_ent(0, 0)

    @pl.when(sid < 2)
    def _():
        pltpu.make_async_copy(r1_hbm, r1_sh, ssem).wait()

    plsc.subcore_barrier()
    fire_rel(0, 0)

    def pair_body(p, carry):
        c0 = p * 2
        fire(c0 + 1, 1)
        drain(0, c0)
        compute(c0, 0)

        @pl.when(p < NCHUNK // 2 - 1)
        def _():
            fire(c0 + 2, 0)

        drain(1, c0 + 1)
        compute(c0 + 1, 1)
        return carry

    lax.fori_loop(0, NCHUNK // 2, pair_body, 0)
    pltpu.sync_copy(sc_v, out_hbm.at[pl.ds(base, BPW)])


def kernel(entity_embedding1, entity_embedding2, relation_embedding1,
           relation_embedding2, heads, relations, tails):
    return _sc_complex_score(
        entity_embedding1, entity_embedding2,
        relation_embedding1, relation_embedding2,
        heads.astype(jnp.int32), relations.astype(jnp.int32),
        tails.astype(jnp.int32))


# speedup vs baseline: 1.0491x; 1.0491x over previous
import functools

import jax
import jax.numpy as jnp
from jax import lax
from jax.experimental import pallas as pl
from jax.experimental.pallas import tpu as pltpu
from jax.experimental.pallas import tpu_sc as plsc

D = 128
B = 16384
NC = 2
NS = 16
NW = NC * NS
BPW = B // NW
CH = 64
NCHUNK = BPW // CH
NSL = D // 16
NR = 1000

_GATHER_DNUMS = lax.GatherDimensionNumbers(
    offset_dims=(), collapsed_slice_dims=(0,), start_index_map=(0,))


def _lane_permute(v, idx):
    return lax.gather(v, idx[:, None], _GATHER_DNUMS, slice_sizes=(1,),
                      mode=lax.GatherScatterMode.PROMISE_IN_BOUNDS)


@functools.partial(
    pl.kernel,
    out_type=jax.ShapeDtypeStruct((B,), jnp.float32),
    mesh=plsc.VectorSubcoreMesh(core_axis_name="c", subcore_axis_name="s"),
    scratch_types=[
        pltpu.VMEM((BPW,), jnp.int32),
        pltpu.VMEM((NCHUNK, CH), jnp.int32),
        pltpu.VMEM((BPW,), jnp.int32),
        pltpu.VMEM((2, CH, D), jnp.float32),
        pltpu.VMEM((2, CH, D), jnp.float32),
        pltpu.VMEM((2, CH, D), jnp.float32),
        pltpu.VMEM((2, CH, D), jnp.float32),
        pltpu.VMEM((2, CH, D), jnp.float32),
        pltpu.VMEM((2, CH, D), jnp.float32),
        pltpu.VMEM((BPW,), jnp.float32),
        pltpu.VMEM_SHARED((NR, D), jnp.float32),
        pltpu.VMEM_SHARED((NR, D), jnp.float32),
        pltpu.SemaphoreType.DMA,
        pltpu.SemaphoreType.DMA,
        pltpu.SemaphoreType.DMA,
        pltpu.SemaphoreType.DMA,
        pltpu.SemaphoreType.DMA,
        pltpu.SemaphoreType.DMA,
    ],
)
def _sc_complex_score(e1_hbm, e2_hbm, r1_hbm, r2_hbm,
                      heads_hbm, rels_hbm, tails_hbm, out_hbm,
                      h_v, r_v, t_v, e11, e12, e21, e22, rr1, rr2,
                      sc_v, r1_sh, r2_sh, sem0, sem1, sem2, sem3,
                      isem, ssem):
    wid = lax.axis_index("s") * NC + lax.axis_index("c")
    sid = lax.axis_index("s")
    base = wid * BPW

    @pl.when(sid == 0)
    def _():
        pltpu.async_copy(r1_hbm, r1_sh, ssem)

    @pl.when(sid == 1)
    def _():
        pltpu.async_copy(r2_hbm, r2_sh, ssem)

    icps = [
        pltpu.async_copy(heads_hbm.at[pl.ds(base, BPW)], h_v, isem),
        pltpu.async_copy(tails_hbm.at[pl.ds(base, BPW)], t_v, isem),
    ]
    icps += [
        pltpu.async_copy(rels_hbm.at[pl.ds(base + cc * CH, CH)], r_v.at[cc],
                         isem)
        for cc in range(NCHUNK)
    ]
    for cp in icps:
        cp.wait()

    sems = (sem0, sem1)
    rsems = (sem2, sem3)

    def fire_ent(c, slot):
        off = c * CH
        hs = h_v.at[pl.ds(off, CH)]
        ts = t_v.at[pl.ds(off, CH)]
        sem = sems[slot]
        pltpu.async_copy(e1_hbm.at[hs], e11.at[slot], sem)
        pltpu.async_copy(e2_hbm.at[hs], e12.at[slot], sem)
        pltpu.async_copy(e1_hbm.at[ts], e21.at[slot], sem)
        pltpu.async_copy(e2_hbm.at[ts], e22.at[slot], sem)

    def fire_rel(c, slot):
        rs = r_v.at[c]
        sem = rsems[slot]
        pltpu.async_copy(r1_sh.at[rs], rr1.at[slot], sem)
        pltpu.async_copy(r2_sh.at[rs], rr2.at[slot], sem)

    def fire(c, slot):
        fire_ent(c, slot)
        fire_rel(c, slot)

    def drain(slot, c):
        off = c * CH
        sem = sems[slot]
        for dst in (e11, e12, e21, e22):
            pltpu.make_async_copy(
                e1_hbm.at[pl.ds(0, CH)], dst.at[slot], sem).wait()
        rs = r_v.at[c]
        rsem = rsems[slot]
        pltpu.make_async_copy(r1_sh.at[rs], rr1.at[slot], rsem).wait()
        pltpu.make_async_copy(r2_sh.at[rs], rr2.at[slot], rsem).wait()

    def compute(c, slot):
        off = c * CH
        b11, b12 = e11.at[slot], e12.at[slot]
        b21, b22 = e21.at[slot], e22.at[slot]
        c1, c2 = rr1.at[slot], rr2.at[slot]

        def group_body(g, rc):
            lane = lax.iota(jnp.int32, 16)
            perms = [lane ^ k for k in (1, 2, 4, 8)]

            @plsc.parallel_loop(0, 16, 1, unroll=4,
                                carry=jnp.zeros((16,), jnp.float32))
            def w_loop(j, w):
                i = g * 16 + j
                accp = jnp.zeros((16,), jnp.float32)
                accq = jnp.zeros((16,), jnp.float32)
                for s in range(NSL):
                    sl = pl.ds(s * 16, 16)
                    a = b11[i, sl]
                    b = b12[i, sl]
                    u = b21[i, sl]
                    v = b22[i, sl]
                    p1 = c1[i, sl]
                    p2 = c2[i, sl]
                    accp = accp + p1 * (a * u + b * v)
                    accq = accq + p2 * (a * v - b * u)
                t = accp + accq
                for p in perms:
                    t = t + _lane_permute(t, p)
                return jnp.where(lane == j, t, w)

            sc_v[pl.ds(off + g * 16, 16)] = 1.0 / (1.0 + jnp.exp(-w_loop))
            return rc

        lax.fori_loop(0, CH // 16, group_body, 0)

    fire_ent(0, 0)

    @pl.when(sid < 2)
    def _():
        pltpu.make_async_copy(r1_hbm, r1_sh, ssem).wait()

    plsc.subcore_barrier()
    fire_rel(0, 0)

    def pair_body(p, carry):
        c0 = p * 2
        fire(c0 + 1, 1)
        drain(0, c0)
        compute(c0, 0)

        @pl.when(p < NCHUNK // 2 - 1)
        def _():
            fire(c0 + 2, 0)

        drain(1, c0 + 1)
        compute(c0 + 1, 1)
        return carry

    lax.fori_loop(0, NCHUNK // 2, pair_body, 0)
    pltpu.sync_copy(sc_v, out_hbm.at[pl.ds(base, BPW)])


def kernel(entity_embedding1, entity_embedding2, relation_embedding1,
           relation_embedding2, heads, relations, tails):
    return _sc_complex_score(
        entity_embedding1, entity_embedding2,
        relation_embedding1, relation_embedding2,
        heads.astype(jnp.int32), relations.astype(jnp.int32),
        tails.astype(jnp.int32))
